# baseline (speedup 1.0000x reference)
import jax
import jax.numpy as jnp
from jax import lax
from jax.experimental import pallas as pl
from jax.experimental.pallas import tpu as pltpu

def _deint_mat():
    rows = lax.broadcasted_iota(jnp.int32, (34, 34), 0)
    cols = lax.broadcasted_iota(jnp.int32, (34, 34), 1)
    ind_x = 1 - jnp.minimum(jnp.abs(cols - 2 * rows), 1)
    ind_y = 1 - jnp.minimum(jnp.abs(cols - 2 * rows + 33), 1)
    is_lt17 = jnp.minimum(jnp.maximum(17 - rows, 0), 1)
    return (ind_x * is_lt17 + ind_y * (1 - is_lt17)).astype(jnp.float32)

_S0 = (0, 5, 6, 11, 12)
_S1 = (7, 8, 13, 14)
_S2 = (9, 10, 15, 16)
_SUBSETS = (_S0, _S1, _S2)
_CROSS_COUNT = (float(len(_S1)), float(len(_S0) + len(_S2)), float(len(_S1)))
_NJ13 = sum(len(s) for s in _SUBSETS)


def _body(k34_ref, s_ref, W0_ref, W1_ref, W2_ref, b0_ref, b1_ref, b2_ref,
          We_ref, be_ref, Wp_ref, bp_ref, out_ref):
    bf16 = jnp.bfloat16
    f32 = jnp.float32
    nb = k34_ref.shape[0]

    t34 = jnp.transpose(k34_ref[...])
    st = jnp.transpose(s_ref[...])
    kxy = jnp.dot(_deint_mat(), t34,
                  preferred_element_type=f32)
    kxt = kxy[0:17]
    kyt = kxy[17:34]
    xmin = jnp.min(kxt, axis=0, keepdims=True)
    xmax = jnp.max(kxt, axis=0, keepdims=True)
    ymin = jnp.min(kyt, axis=0, keepdims=True)
    ymax = jnp.max(kyt, axis=0, keepdims=True)
    xn = ((kxt - xmin) / (xmax - xmin + 1e-6)).astype(bf16)
    yn = ((kyt - ymin) / (ymax - ymin + 1e-6)).astype(bf16)
    stb = st.astype(bf16)
    ones_row = jnp.ones((1, nb), bf16)

    w_refs = (W0_ref, W1_ref, W2_ref)
    b_refs = (b0_ref, b1_ref, b2_ref)
    srow_sel = jnp.concatenate(
        [jnp.zeros((64, 2), bf16), jnp.ones((64, 1), bf16),
         jnp.zeros((64, 1), bf16)], axis=1)
    totals = []
    feats = []
    for li, S in enumerate(_SUBSETS):
        p = jnp.concatenate(
            [jnp.concatenate([xn[u:u + 1], yn[u:u + 1], stb[u:u + 1],
                              ones_row], axis=0) for u in S],
            axis=1)
        wt = jnp.concatenate(
            [jnp.transpose(w_refs[li][...]),
             jnp.transpose(b_refs[li][...])], axis=1).astype(bf16)
        wcat = jnp.concatenate([wt, srow_sel], axis=0)
        hs = jnp.dot(wcat, p,
                     preferred_element_type=f32).astype(bf16)
        ms = []
        sbs = []
        for idx in range(len(S)):
            blk = hs[:, idx * nb:(idx + 1) * nb]
            sb = blk[64:128]
            ms.append(jnp.maximum(blk[0:64], 0.0) * sb)
            sbs.append(sb)
        total = ms[0]
        for mm in ms[1:]:
            total = total + mm
        totals.append(total)
        for idx in range(len(S)):
            feats.append((total - ms[idx]) * sbs[idx])

    fcat = jnp.concatenate(feats, axis=1)
    fcat = jnp.concatenate(
        [fcat, jnp.ones((1, _NJ13 * nb), bf16)], axis=0)
    wea = We_ref[0:64, :]
    web = We_ref[64:128, :]
    bet = jnp.transpose(be_ref[...])
    weabt = jnp.concatenate([
        jnp.concatenate([jnp.transpose(wea - web), bet], axis=1),
        jnp.concatenate([jnp.transpose(web), jnp.zeros((64, 1), f32)],
                        axis=1)], axis=0).astype(bf16)
    uv = jnp.dot(weabt, fcat,
                 preferred_element_type=f32).astype(bf16)
    betb = bet.astype(bf16)

    pooled_parts = []
    off = 0
    for li, S in enumerate(_SUBSETS):
        n = len(S)
        us = [uv[0:64, (off + idx) * nb:(off + idx + 1) * nb]
              for idx in range(n)]
        vs = [uv[64:128, (off + idx) * nb:(off + idx + 1) * nb]
              for idx in range(n)]
        off += n
        maxall = vs[0]
        for v in vs[1:]:
            maxall = jnp.maximum(maxall, v)
        zsum = None
        for idx in range(n):
            om = None
            for j2 in range(n):
                if j2 == idx:
                    continue
                om = vs[j2] if om is None else jnp.maximum(om, vs[j2])
            z = jnp.maximum(us[idx] + jnp.maximum(om, 0.0), 0.0)
            zsum = z if zsum is None else zsum + z
        zc = jnp.maximum(betb + maxall, 0.0)
        mean_z = (zsum + _CROSS_COUNT[li] * zc) * (1.0 / 17.0)
        mean_h = totals[li] * ((n - 1) / 17.0)
        pooled_parts.append(mean_h)
        pooled_parts.append(mean_z)

    poolt = jnp.concatenate(pooled_parts, axis=0)
    out = lax.dot_general(poolt, Wp_ref[...].astype(bf16),
                          (((0,), (0,)), ((), ())),
                          preferred_element_type=f32)
    out_ref[...] = out + bp_ref[...]


def kernel(keypoints, scores, W0, b0, W1, b1, W2, b2, We, be, Wp, bp):
    n = keypoints.shape[0]
    k34 = keypoints.reshape(n, 34)
    block_n = min(n, 2048)
    grid = (n // block_n,)
    rep = lambda i: (0, 0)
    return pl.pallas_call(
        _body,
        grid=grid,
        in_specs=[
            pl.BlockSpec((block_n, 34), lambda i: (i, 0)),
            pl.BlockSpec((block_n, 17), lambda i: (i, 0)),
            pl.BlockSpec((3, 64), rep),
            pl.BlockSpec((3, 64), rep),
            pl.BlockSpec((3, 64), rep),
            pl.BlockSpec((1, 64), rep),
            pl.BlockSpec((1, 64), rep),
            pl.BlockSpec((1, 64), rep),
            pl.BlockSpec((128, 64), rep),
            pl.BlockSpec((1, 64), rep),
            pl.BlockSpec((384, 128), rep),
            pl.BlockSpec((1, 128), rep),
        ],
        out_specs=pl.BlockSpec((block_n, 128), lambda i: (i, 0)),
        out_shape=jax.ShapeDtypeStruct((n, 128), jnp.float32),
        compiler_params=pltpu.CompilerParams(
            dimension_semantics=("parallel",),
            disable_bounds_checks=True,
            skip_device_barrier=True),
    )(k34, scores, W0, W1, W2, b0.reshape(1, 64), b1.reshape(1, 64),
      b2.reshape(1, 64), We, be.reshape(1, 64), Wp, bp.reshape(1, 128))


# speedup vs baseline: 1.0233x; 1.0233x over previous
import jax
import jax.numpy as jnp
from jax import lax
from jax.experimental import pallas as pl
from jax.experimental.pallas import tpu as pltpu

def _deint_mat():
    rows = lax.broadcasted_iota(jnp.int32, (34, 34), 0)
    cols = lax.broadcasted_iota(jnp.int32, (34, 34), 1)
    ind_x = 1 - jnp.minimum(jnp.abs(cols - 2 * rows), 1)
    ind_y = 1 - jnp.minimum(jnp.abs(cols - 2 * rows + 33), 1)
    is_lt17 = jnp.minimum(jnp.maximum(17 - rows, 0), 1)
    return (ind_x * is_lt17 + ind_y * (1 - is_lt17)).astype(jnp.float32)

_S0 = (0, 5, 6, 11, 12)
_S1 = (7, 8, 13, 14)
_S2 = (9, 10, 15, 16)
_SUBSETS = (_S0, _S1, _S2)
_CROSS_COUNT = (float(len(_S1)), float(len(_S0) + len(_S2)), float(len(_S1)))
_NJ13 = sum(len(s) for s in _SUBSETS)


def _body(k34_ref, s_ref, W0_ref, W1_ref, W2_ref, b0_ref, b1_ref, b2_ref,
          We_ref, be_ref, Wp_ref, bp_ref, out_ref):
    bf16 = jnp.bfloat16
    f32 = jnp.float32
    nb = k34_ref.shape[0]

    t34 = jnp.transpose(k34_ref[...])
    st = jnp.transpose(s_ref[...])
    kxy = jnp.dot(_deint_mat(), t34,
                  preferred_element_type=f32)
    kxt = kxy[0:17]
    kyt = kxy[17:34]
    xmin = jnp.min(kxt, axis=0, keepdims=True)
    xmax = jnp.max(kxt, axis=0, keepdims=True)
    ymin = jnp.min(kyt, axis=0, keepdims=True)
    ymax = jnp.max(kyt, axis=0, keepdims=True)
    xn = ((kxt - xmin) / (xmax - xmin + 1e-6)).astype(bf16)
    yn = ((kyt - ymin) / (ymax - ymin + 1e-6)).astype(bf16)
    stb = st.astype(bf16)
    ones_row = jnp.ones((1, nb), bf16)

    w_refs = (W0_ref, W1_ref, W2_ref)
    b_refs = (b0_ref, b1_ref, b2_ref)
    srow_sel = jnp.concatenate(
        [jnp.concatenate([jnp.zeros((1, 2), bf16), jnp.ones((1, 1), bf16),
                          jnp.zeros((1, 1), bf16)], axis=1),
         jnp.zeros((7, 4), bf16)], axis=0)
    totals = []
    feats = []
    for li, S in enumerate(_SUBSETS):
        p = jnp.concatenate(
            [jnp.concatenate([xn[u:u + 1], yn[u:u + 1], stb[u:u + 1],
                              ones_row], axis=0) for u in S],
            axis=1)
        wt = jnp.concatenate(
            [jnp.transpose(w_refs[li][...]),
             jnp.transpose(b_refs[li][...])], axis=1).astype(bf16)
        wcat = jnp.concatenate([wt, srow_sel], axis=0)
        hs = jnp.dot(wcat, p,
                     preferred_element_type=f32).astype(bf16)
        ms = []
        sbs = []
        for idx in range(len(S)):
            blk = hs[:, idx * nb:(idx + 1) * nb]
            sb = blk[64:65]
            ms.append(jnp.maximum(blk[0:64], 0.0) * sb)
            sbs.append(sb)
        total = ms[0]
        for mm in ms[1:]:
            total = total + mm
        totals.append(total)
        for idx in range(len(S)):
            feats.append((total - ms[idx]) * sbs[idx])

    fcat = jnp.concatenate(feats, axis=1)
    fcat = jnp.concatenate(
        [fcat, jnp.ones((1, _NJ13 * nb), bf16)], axis=0)
    wea = We_ref[0:64, :]
    web = We_ref[64:128, :]
    bet = jnp.transpose(be_ref[...])
    weabt = jnp.concatenate([
        jnp.concatenate([jnp.transpose(wea - web), bet], axis=1),
        jnp.concatenate([jnp.transpose(web), jnp.zeros((64, 1), f32)],
                        axis=1)], axis=0).astype(bf16)
    uv = jnp.dot(weabt, fcat,
                 preferred_element_type=f32).astype(bf16)
    betb = bet.astype(bf16)

    pooled_parts = []
    off = 0
    for li, S in enumerate(_SUBSETS):
        n = len(S)
        us = [uv[0:64, (off + idx) * nb:(off + idx + 1) * nb]
              for idx in range(n)]
        vs = [uv[64:128, (off + idx) * nb:(off + idx + 1) * nb]
              for idx in range(n)]
        off += n
        maxall = vs[0]
        for v in vs[1:]:
            maxall = jnp.maximum(maxall, v)
        zsum = None
        for idx in range(n):
            om = None
            for j2 in range(n):
                if j2 == idx:
                    continue
                om = vs[j2] if om is None else jnp.maximum(om, vs[j2])
            z = jnp.maximum(us[idx] + jnp.maximum(om, 0.0), 0.0)
            zsum = z if zsum is None else zsum + z
        zc = jnp.maximum(betb + maxall, 0.0)
        mean_z = (zsum + _CROSS_COUNT[li] * zc) * (1.0 / 17.0)
        mean_h = totals[li] * ((n - 1) / 17.0)
        pooled_parts.append(mean_h)
        pooled_parts.append(mean_z)

    poolt = jnp.concatenate(pooled_parts, axis=0)
    out = lax.dot_general(poolt, Wp_ref[...].astype(bf16),
                          (((0,), (0,)), ((), ())),
                          preferred_element_type=f32)
    out_ref[...] = out + bp_ref[...]


def kernel(keypoints, scores, W0, b0, W1, b1, W2, b2, We, be, Wp, bp):
    n = keypoints.shape[0]
    k34 = keypoints.reshape(n, 34)
    block_n = min(n, 2048)
    grid = (n // block_n,)
    rep = lambda i: (0, 0)
    return pl.pallas_call(
        _body,
        grid=grid,
        in_specs=[
            pl.BlockSpec((block_n, 34), lambda i: (i, 0)),
            pl.BlockSpec((block_n, 17), lambda i: (i, 0)),
            pl.BlockSpec((3, 64), rep),
            pl.BlockSpec((3, 64), rep),
            pl.BlockSpec((3, 64), rep),
            pl.BlockSpec((1, 64), rep),
            pl.BlockSpec((1, 64), rep),
            pl.BlockSpec((1, 64), rep),
            pl.BlockSpec((128, 64), rep),
            pl.BlockSpec((1, 64), rep),
            pl.BlockSpec((384, 128), rep),
            pl.BlockSpec((1, 128), rep),
        ],
        out_specs=pl.BlockSpec((block_n, 128), lambda i: (i, 0)),
        out_shape=jax.ShapeDtypeStruct((n, 128), jnp.float32),
        compiler_params=pltpu.CompilerParams(
            dimension_semantics=("parallel",),
            disable_bounds_checks=True,
            skip_device_barrier=True),
    )(k34, scores, W0, W1, W2, b0.reshape(1, 64), b1.reshape(1, 64),
      b2.reshape(1, 64), We, be.reshape(1, 64), Wp, bp.reshape(1, 128))
